# baseline (speedup 1.0000x reference)
import functools

import jax
import jax.numpy as jnp
from jax import lax
from jax.experimental import pallas as pl
from jax.experimental.pallas import tpu as pltpu
from jax.experimental.pallas import tpu_sc as plsc

_R_MIN = 1.0
_R_MAX = 5.0
_L = 16
_NW = 32


def _tree_sum(vs):
    vs = list(vs)
    while len(vs) > 1:
        nxt = [vs[i] + vs[i + 1] for i in range(0, len(vs) - 1, 2)]
        if len(vs) % 2:
            nxt.append(vs[-1])
        vs = nxt
    return vs[0]


def _fast_rsqrt(x):
    bits = lax.bitcast_convert_type(x, jnp.int32)
    seed = jnp.int32(0x5F3759DF) - lax.shift_right_logical(bits, 1)
    y = lax.bitcast_convert_type(seed, jnp.float32)
    for _ in range(3):
        y = y * (1.5 - 0.5 * x * y * y)
    return y


@functools.lru_cache(maxsize=None)
def _make_sc_kernel(b_per_w):
    mesh = plsc.VectorSubcoreMesh(core_axis_name="c", subcore_axis_name="s")
    num_cores = 2
    n_pass = b_per_w // _L
    @functools.partial(
        pl.kernel,
        mesh=mesh,
        compiler_params=pltpu.CompilerParams(
            needs_layout_passes=False, use_tc_tiling_on_sc=True),
        out_type=(
            jax.ShapeDtypeStruct((_NW, b_per_w), jnp.float32),
            jax.ShapeDtypeStruct((_NW, _L), jnp.float32),
        ),
        scratch_types=[
            pltpu.VMEM((b_per_w,), jnp.int32),
            pltpu.VMEM((b_per_w,), jnp.int32),
            pltpu.VMEM((_L, _L, 128), jnp.float32),
            pltpu.VMEM((_L, _L, 128), jnp.float32),
            pltpu.VMEM((b_per_w,), jnp.float32),
            pltpu.VMEM((b_per_w,), jnp.float32),
            pltpu.VMEM((_L,), jnp.float32),
            pltpu.SemaphoreType.DMA,
            pltpu.SemaphoreType.DMA,
        ],
    )
    def mf_kernel(uw_hbm, iw_hbm, rat_hbm, u_hbm, it_hbm, dum_hbm, out_hbm,
                  err_hbm, uidx_v, iidx_v, ubuf, ibuf, rat_v,
                  out_v, err_v, usem, isem):
        wid = lax.axis_index("s") * num_cores + lax.axis_index("c")

        pltpu.sync_copy(u_hbm.at[wid], uidx_v)
        pltpu.sync_copy(it_hbm.at[wid], iidx_v)
        pltpu.sync_copy(rat_hbm.at[wid], rat_v)

        lane = lax.iota(jnp.int32, _L)
        err_v[...] = jnp.zeros((_L,), jnp.float32)
        inv_l = 1.0 / _L
        inv_span = 1.0 / (_R_MAX - _R_MIN)

        def one_pass(p, carry):
            e0 = p * _L

            vu = uidx_v[pl.ds(e0, _L)]
            vi = iidx_v[pl.ds(e0, _L)]
            cu_vec = vu - (vu & 127)
            ci_vec = vi - (vi & 127)
            for j in range(_L):
                c_u = pl.multiple_of(cu_vec[j], 128)
                pltpu.async_copy(
                    uw_hbm.at[:, pl.ds(c_u, 128)], ubuf.at[j], usem)
                c_i = pl.multiple_of(ci_vec[j], 128)
                pltpu.async_copy(
                    iw_hbm.at[:, pl.ds(c_i, 128)], ibuf.at[j], isem)
            pltpu.make_async_copy(dum_hbm, ubuf, usem).wait()
            pltpu.make_async_copy(dum_hbm, ibuf, isem).wait()

            rr_u = uidx_v[pl.ds(e0, _L)] & 127
            rr_i = iidx_v[pl.ds(e0, _L)] & 127
            u = [plsc.load_gather(ubuf, [lane, (lane + f) & (_L - 1), rr_u])
                 for f in range(_L)]
            mu = _tree_sum(u) * inv_l
            u = [uf - mu for uf in u]
            snu = _tree_sum([uf * uf for uf in u])
            it = [plsc.load_gather(ibuf, [lane, (lane + f) & (_L - 1), rr_i])
                  for f in range(_L)]
            mi = _tree_sum(it) * inv_l
            it = [x - mi for x in it]
            sni = _tree_sum([x * x for x in it])
            dot = _tree_sum([u[f] * it[f] for f in range(_L)])
            rs_u = _fast_rsqrt(jnp.maximum(snu, 1e-24))
            rs_i = _fast_rsqrt(jnp.maximum(sni, 1e-24))
            mf = dot * rs_u * rs_i
            out_v[pl.ds(e0, _L)] = mf * (_R_MAX - _R_MIN) + _R_MIN
            d = mf - (rat_v[pl.ds(e0, _L)] - _R_MIN) * inv_span
            err_v[...] = err_v[...] + d * d
            return carry

        lax.fori_loop(0, n_pass, one_pass, 0)

        pltpu.sync_copy(out_v, out_hbm.at[wid])
        pltpu.sync_copy(err_v, err_hbm.at[wid])

    return mf_kernel


def kernel(user_weight, item_weight, rating, user, item):
    batch = user.shape[0]
    b_per_w = batch // _NW
    sc = _make_sc_kernel(b_per_w)
    uw_t = user_weight.T
    iw_t = item_weight.T
    u2 = user.astype(jnp.int32).reshape(_NW, b_per_w)
    i2 = item.astype(jnp.int32).reshape(_NW, b_per_w)
    r2 = rating.reshape(_NW, b_per_w)
    dummy = jnp.zeros((_L, _L, 128), jnp.float32)
    preds, err = sc(uw_t, iw_t, r2, u2, i2, dummy)
    target_rating = preds.reshape(batch)
    loss = jnp.sum(err) * (1.0 / batch)
    return (loss, target_rating)


# speedup vs baseline: 1.1148x; 1.1148x over previous
import functools

import jax
import jax.numpy as jnp
from jax import lax
from jax.experimental import pallas as pl
from jax.experimental.pallas import tpu as pltpu
from jax.experimental.pallas import tpu_sc as plsc

_R_MIN = 1.0
_R_MAX = 5.0
_L = 16
_NW = 32


def _tree_sum(vs):
    vs = list(vs)
    while len(vs) > 1:
        nxt = [vs[i] + vs[i + 1] for i in range(0, len(vs) - 1, 2)]
        if len(vs) % 2:
            nxt.append(vs[-1])
        vs = nxt
    return vs[0]


def _fast_rsqrt(x):
    bits = lax.bitcast_convert_type(x, jnp.int32)
    seed = jnp.int32(0x5F3759DF) - lax.shift_right_logical(bits, 1)
    y = lax.bitcast_convert_type(seed, jnp.float32)
    for _ in range(3):
        y = y * (1.5 - 0.5 * x * y * y)
    return y


@functools.lru_cache(maxsize=None)
def _make_sc_kernel(b_per_w):
    mesh = plsc.VectorSubcoreMesh(core_axis_name="c", subcore_axis_name="s")
    num_cores = 2
    @functools.partial(
        pl.kernel,
        mesh=mesh,
        compiler_params=pltpu.CompilerParams(
            needs_layout_passes=False, use_tc_tiling_on_sc=True),
        out_type=(
            jax.ShapeDtypeStruct((_NW, b_per_w), jnp.float32),
            jax.ShapeDtypeStruct((_NW, _L), jnp.float32),
        ),
        scratch_types=[
            pltpu.VMEM((b_per_w + _L,), jnp.int32),
            pltpu.VMEM((b_per_w + _L,), jnp.int32),
            pltpu.VMEM((_L, _L, 128), jnp.float32),
            pltpu.VMEM((_L, _L, 128), jnp.float32),
            pltpu.VMEM((b_per_w + _L,), jnp.float32),
            pltpu.VMEM((b_per_w + _L,), jnp.float32),
            pltpu.VMEM((_L,), jnp.float32),
            pltpu.SemaphoreType.DMA,
            pltpu.SemaphoreType.DMA,
        ],
    )
    def mf_kernel(uw_hbm, iw_hbm, rat_hbm, u_hbm, it_hbm, dum_hbm, out_hbm,
                  err_hbm, uidx_v, iidx_v, ubuf, ibuf, rat_v,
                  out_v, err_v, usem, isem):
        wid = lax.axis_index("s") * num_cores + lax.axis_index("c")

        pltpu.sync_copy(u_hbm.at[wid], uidx_v.at[pl.ds(0, b_per_w)])
        pltpu.sync_copy(it_hbm.at[wid], iidx_v.at[pl.ds(0, b_per_w)])
        pltpu.sync_copy(rat_hbm.at[wid], rat_v.at[pl.ds(0, b_per_w)])

        lane = lax.iota(jnp.int32, _L)
        err_v[...] = jnp.zeros((_L,), jnp.float32)
        inv_l = 1.0 / _L
        inv_span = 1.0 / (_R_MAX - _R_MIN)

        half = _L // 2
        n_half = b_per_w // half
        max_off = b_per_w - half

        def enq_half(h, s):
            off = jnp.minimum(h * half, max_off)
            vu = uidx_v[pl.ds(off, _L)]
            vi = iidx_v[pl.ds(off, _L)]
            cu_vec = vu - (vu & 127)
            ci_vec = vi - (vi & 127)
            for j in range(half):
                c_u = pl.multiple_of(cu_vec[j], 128)
                pltpu.async_copy(
                    uw_hbm.at[:, pl.ds(c_u, 128)], ubuf.at[s * half + j], usem)
                c_i = pl.multiple_of(ci_vec[j], 128)
                pltpu.async_copy(
                    iw_hbm.at[:, pl.ds(c_i, 128)], ibuf.at[s * half + j], isem)

        def drain_half():
            pltpu.make_async_copy(dum_hbm, ubuf.at[pl.ds(0, half)], usem).wait()
            pltpu.make_async_copy(dum_hbm, ibuf.at[pl.ds(0, half)], isem).wait()

        def compute_half(k, s):
            e0 = k * _L + s * half
            slot = s * half + (lane & (half - 1))
            rr_u = uidx_v[pl.ds(e0, _L)] & 127
            rr_i = iidx_v[pl.ds(e0, _L)] & 127
            u = [plsc.load_gather(ubuf, [slot, (lane + f) & (_L - 1), rr_u])
                 for f in range(_L)]
            mu = _tree_sum(u) * inv_l
            u = [uf - mu for uf in u]
            snu = _tree_sum([uf * uf for uf in u])
            it = [plsc.load_gather(ibuf, [slot, (lane + f) & (_L - 1), rr_i])
                  for f in range(_L)]
            mi = _tree_sum(it) * inv_l
            it = [x - mi for x in it]
            sni = _tree_sum([x * x for x in it])
            dot = _tree_sum([u[f] * it[f] for f in range(_L)])
            rs_u = _fast_rsqrt(jnp.maximum(snu, 1e-24))
            rs_i = _fast_rsqrt(jnp.maximum(sni, 1e-24))
            mf = dot * rs_u * rs_i
            keep = lane < half
            old = out_v[pl.ds(e0, _L)]
            out_v[pl.ds(e0, _L)] = jnp.where(
                keep, mf * (_R_MAX - _R_MIN) + _R_MIN, old)
            d = mf - (rat_v[pl.ds(e0, _L)] - _R_MIN) * inv_span
            err_v[...] = err_v[...] + jnp.where(keep, d * d, 0.0)

        def pair(k, carry):
            enq_half(2 * k + 1, 1)
            drain_half()
            compute_half(k, 0)
            enq_half(2 * k + 2, 0)
            drain_half()
            compute_half(k, 1)
            return carry

        enq_half(0, 0)
        lax.fori_loop(0, n_half // 2, pair, 0)
        drain_half()

        pltpu.sync_copy(out_v.at[pl.ds(0, b_per_w)], out_hbm.at[wid])
        pltpu.sync_copy(err_v, err_hbm.at[wid])

    return mf_kernel


def kernel(user_weight, item_weight, rating, user, item):
    batch = user.shape[0]
    b_per_w = batch // _NW
    sc = _make_sc_kernel(b_per_w)
    uw_t = user_weight.T
    iw_t = item_weight.T
    u2 = user.astype(jnp.int32).reshape(_NW, b_per_w)
    i2 = item.astype(jnp.int32).reshape(_NW, b_per_w)
    r2 = rating.reshape(_NW, b_per_w)
    dummy = jnp.zeros((_L // 2, _L, 128), jnp.float32)
    preds, err = sc(uw_t, iw_t, r2, u2, i2, dummy)
    target_rating = preds.reshape(batch)
    loss = jnp.sum(err) * (1.0 / batch)
    return (loss, target_rating)
